# Initial kernel scaffold; baseline (speedup 1.0000x reference)
import jax
import jax.numpy as jnp
from jax.experimental import pallas as pl


def kernel(atom_type_emb, w_frac, rbf_offset, atom_types, frac_coords, cell, batch_idx, num_atoms, token_idx, edge_index, edge_distance, edge_distance_vec):
    raise NotImplementedError("write your pallas kernel here")


# speedup vs baseline: 1.4655x; 1.4655x over previous
import functools
import math

import jax
import jax.numpy as jnp
from jax.experimental import pallas as pl
from jax.experimental.pallas import tpu as pltpu


def _round_up(x, m):
    return ((x + m - 1) // m) * m


_COMPILER_PARAMS = pltpu.CompilerParams(
    dimension_semantics=("parallel",),
    vmem_limit_bytes=64 * 1024 * 1024,
)


def _edge_kernel(dist_ref, dvec_ref, cn_ref, offset_ref, out_ref, *, coeff, num_g):
    d = dist_ref[...]
    v = dvec_ref[...]
    cn = cn_ref[...]

    u = v / d
    ip0 = cn[:, 0:1] * u[:, 0:1] + cn[:, 1:2] * u[:, 1:2] + cn[:, 2:3] * u[:, 2:3]
    ip1 = cn[:, 3:4] * u[:, 0:1] + cn[:, 4:5] * u[:, 1:2] + cn[:, 5:6] * u[:, 2:3]
    ip2 = cn[:, 6:7] * u[:, 0:1] + cn[:, 7:8] * u[:, 1:2] + cn[:, 8:9] * u[:, 2:3]
    ip = jnp.concatenate([ip0, ip1, ip2], axis=1)
    ip = jnp.clip(ip, -1.0, 1.0)

    cos_a = -ip
    sin_a = jnp.sqrt(jnp.maximum(1.0 - ip * ip, 0.0))
    cos_a = jnp.where(jnp.isfinite(cos_a), cos_a, 0.0)
    sin_a = jnp.where(jnp.isfinite(sin_a), sin_a, 0.0)

    diff = d - offset_ref[...]
    out_ref[:, 0:num_g] = jnp.exp(coeff * diff * diff)
    out_ref[:, num_g:num_g + 3] = cos_a
    out_ref[:, num_g + 3:num_g + 6] = sin_a


def _node_kernel(types_ref, frac_ref, tab_ref, w_ref, out_ref, *, c):
    t = types_ref[...]
    rows = tab_ref.shape[0]
    iota = jax.lax.broadcasted_iota(jnp.int32, (t.shape[0], rows), 1)
    onehot = (iota == t).astype(jnp.float32)
    emb = jnp.dot(onehot, tab_ref[...], preferred_element_type=jnp.float32)

    f = frac_ref[...]
    w = w_ref[...]
    emb = (emb
           + f[:, 0:1] * w[0:1, :]
           + f[:, 1:2] * w[1:2, :]
           + f[:, 2:3] * w[2:3, :])
    out_ref[:, 0:c] = emb
    out_ref[:, c:] = jnp.zeros((emb.shape[0], out_ref.shape[1] - c), jnp.float32)


def kernel(atom_type_emb, w_frac, rbf_offset, atom_types, frac_coords, cell,
           batch_idx, num_atoms, token_idx, edge_index, edge_distance,
           edge_distance_vec):
    lmax = 2
    l2 = (lmax + 1) ** 2
    n = atom_types.shape[0]
    c = atom_type_emb.shape[1]
    g = rbf_offset.shape[0]
    e = edge_index.shape[1]
    cutoff = 5.0
    delta = cutoff / (g - 1)
    coeff = -0.5 / (2.0 * delta) ** 2

    cellnorm = cell / jnp.linalg.norm(cell, axis=-1, keepdims=True)
    cn_flat = cellnorm.reshape(cell.shape[0], 9).astype(jnp.float32)
    edge2graph = batch_idx[edge_index[0]]
    cn_pe = cn_flat[edge2graph]

    e_tile = 4096
    e_pad = _round_up(max(e, 1), e_tile)
    dist = edge_distance.reshape(e, 1).astype(jnp.float32)
    dvec = edge_distance_vec.astype(jnp.float32)
    if e_pad != e:
        dist = jnp.pad(dist, ((0, e_pad - e), (0, 0)), constant_values=1.0)
        dvec = jnp.pad(dvec, ((0, e_pad - e), (0, 0)))
        cn_pe = jnp.pad(cn_pe, ((0, e_pad - e), (0, 0)))

    edge_feat = pl.pallas_call(
        functools.partial(_edge_kernel, coeff=float(coeff), num_g=g),
        out_shape=jax.ShapeDtypeStruct((e_pad, g + 6), jnp.float32),
        grid=(e_pad // e_tile,),
        in_specs=[
            pl.BlockSpec((e_tile, 1), lambda i: (i, 0)),
            pl.BlockSpec((e_tile, 3), lambda i: (i, 0)),
            pl.BlockSpec((e_tile, 9), lambda i: (i, 0)),
            pl.BlockSpec((1, g), lambda i: (0, 0)),
        ],
        out_specs=pl.BlockSpec((e_tile, g + 6), lambda i: (i, 0)),
        compiler_params=_COMPILER_PARAMS,
    )(dist, dvec, cn_pe, rbf_offset.reshape(1, g).astype(jnp.float32))
    if e_pad != e:
        edge_feat = edge_feat[:e]

    n_tile = 1024
    n_pad = _round_up(max(n, 1), n_tile)
    tab_rows = _round_up(atom_type_emb.shape[0], 8)
    tab = jnp.pad(atom_type_emb.astype(jnp.float32),
                  ((0, tab_rows - atom_type_emb.shape[0]), (0, 0)))
    types = atom_types.reshape(n, 1).astype(jnp.int32)
    frac = frac_coords.astype(jnp.float32)
    if n_pad != n:
        types = jnp.pad(types, ((0, n_pad - n), (0, 0)))
        frac = jnp.pad(frac, ((0, n_pad - n), (0, 0)))

    x_flat = pl.pallas_call(
        functools.partial(_node_kernel, c=c),
        out_shape=jax.ShapeDtypeStruct((n_pad, l2 * c), jnp.float32),
        grid=(n_pad // n_tile,),
        in_specs=[
            pl.BlockSpec((n_tile, 1), lambda i: (i, 0)),
            pl.BlockSpec((n_tile, 3), lambda i: (i, 0)),
            pl.BlockSpec((tab_rows, c), lambda i: (0, 0)),
            pl.BlockSpec((3, c), lambda i: (0, 0)),
        ],
        out_specs=pl.BlockSpec((n_tile, l2 * c), lambda i: (i, 0)),
        compiler_params=_COMPILER_PARAMS,
    )(types, frac, tab, w_frac.astype(jnp.float32))
    if n_pad != n:
        x_flat = x_flat[:n]

    out = {
        "x": x_flat,
        "num_atoms": num_atoms,
        "batch": batch_idx,
        "token_idx": token_idx,
    }
    return out, edge_feat
